# baseline (speedup 1.0000x reference)
import jax
import jax.numpy as jnp
from jax import lax
from jax.experimental import pallas as pl
from jax.experimental.pallas import tpu as pltpu
from jax.experimental.pallas import tpu_sc as plsc

NODE_NUM = 100000
EMBED_DIM = 128
BATCH = 16384
HIST = 50

NC = 2
NS = 16
NW = NC * NS

TOTAL = BATCH * HIST
PER_W = TOTAL // NW
CHUNK = 128
NCHUNK = PER_W // CHUNK
CPH = BATCH // CHUNK

NPAIR = 3
PREFP = 2
PREF2 = 6
NIDX = 8


def _body(tt_hbm, w_hbm, out_hbm, idx_v, buf_v, isems, gsems, wsems):
    wid = lax.axis_index("s") * NC + lax.axis_index("c")
    base = wid * PER_W
    out_flat = out_hbm.reshape(TOTAL, EMBED_DIM)

    def stage(c):
        gc = wid * NCHUNK + c
        h = lax.div(gc, CPH)
        b0 = lax.rem(gc, CPH) * CHUNK
        s = lax.rem(c, NIDX)
        pltpu.make_async_copy(
            tt_hbm.at[h, pl.ds(b0, CHUNK)], idx_v.at[s], isems.at[s]
        ).start()

    def gather_pair(kp, pp):
        for half in range(2):
            c = kp * 2 + half
            s = lax.rem(c, NIDX)
            pltpu.make_async_copy(
                tt_hbm.at[0, pl.ds(0, CHUNK)], idx_v.at[s], isems.at[s]
            ).wait()
            pltpu.make_async_copy(
                w_hbm.at[idx_v.at[s]],
                buf_v.at[pp, pl.ds(half * CHUNK, CHUNK)],
                gsems.at[pp],
            ).start()

    for c in range(PREF2):
        stage(c)
    for kp in range(PREFP):
        gather_pair(kp, kp)

    def step(k, carry):
        p = lax.rem(k, NPAIR)
        for _ in range(2):
            pltpu.make_async_copy(
                w_hbm.at[idx_v.at[0]], buf_v.at[p, pl.ds(0, CHUNK)], gsems.at[p]
            ).wait()
        pltpu.make_async_copy(
            buf_v.at[p],
            out_flat.at[pl.ds(base + k * 2 * CHUNK, 2 * CHUNK)],
            wsems.at[p],
        ).start()
        kp = k + PREFP

        @pl.when(kp < NCHUNK // 2)
        def _():
            pp = lax.rem(kp, NPAIR)

            @pl.when(kp >= NPAIR)
            def _():
                pltpu.make_async_copy(
                    buf_v.at[pp],
                    out_flat.at[pl.ds(base + (kp - NPAIR) * 2 * CHUNK, 2 * CHUNK)],
                    wsems.at[pp],
                ).wait()

            gather_pair(kp, pp)

        for half in range(2):
            c = k * 2 + PREF2 + half

            @pl.when(c < NCHUNK)
            def _():
                stage(c)

        return carry

    lax.fori_loop(0, NCHUNK // 2, step, 0)

    for t in range(NPAIR):
        k = NCHUNK // 2 - NPAIR + t
        p = k % NPAIR
        pltpu.make_async_copy(
            buf_v.at[p],
            out_flat.at[pl.ds(base + k * 2 * CHUNK, 2 * CHUNK)],
            wsems.at[p],
        ).wait()


@jax.jit
def _run(targ, W):
    tt = targ.T
    mesh = plsc.VectorSubcoreMesh(core_axis_name="c", subcore_axis_name="s")
    k = pl.kernel(
        _body,
        out_type=jax.ShapeDtypeStruct((HIST, BATCH, EMBED_DIM), jnp.float32),
        mesh=mesh,
        compiler_params=pltpu.CompilerParams(use_tc_tiling_on_sc=True),
        scratch_types=[
            pltpu.VMEM((NIDX, CHUNK), jnp.int32),
            pltpu.VMEM((NPAIR, 2 * CHUNK, EMBED_DIM), jnp.float32),
            pltpu.SemaphoreType.DMA((NIDX,)),
            pltpu.SemaphoreType.DMA((NPAIR,)),
            pltpu.SemaphoreType.DMA((NPAIR,)),
        ],
    )
    out_t = k(tt, W)
    return jnp.transpose(out_t, (1, 0, 2))


def kernel(targ, W):
    return _run(targ.astype(jnp.int32), W)


# speedup vs baseline: 1.0051x; 1.0051x over previous
import jax
import jax.numpy as jnp
from jax import lax
from jax.experimental import pallas as pl
from jax.experimental.pallas import tpu as pltpu
from jax.experimental.pallas import tpu_sc as plsc

NODE_NUM = 100000
EMBED_DIM = 128
BATCH = 16384
HIST = 50

NC = 2
NS = 16
NW = NC * NS

TOTAL = BATCH * HIST
PER_W = TOTAL // NW
CHUNK = 128
NCHUNK = PER_W // CHUNK
CPH = BATCH // CHUNK

NBUF = 7
PREF = 4
PREF2 = 6
NIDX = 8


def _body(tt_hbm, w_hbm, out_hbm, idx_v, buf_v, isems, gsems, wsems):
    wid = lax.axis_index("s") * NC + lax.axis_index("c")
    base = wid * PER_W
    out_flat = out_hbm.reshape(TOTAL, EMBED_DIM)

    def stage(c):
        gc = wid * NCHUNK + c
        h = lax.div(gc, CPH)
        b0 = lax.rem(gc, CPH) * CHUNK
        s = lax.rem(c, NIDX)
        pltpu.make_async_copy(
            tt_hbm.at[h, pl.ds(b0, CHUNK)], idx_v.at[s], isems.at[s]
        ).start()

    def gather(g, bg):
        s = lax.rem(g, NIDX)
        pltpu.make_async_copy(
            tt_hbm.at[0, pl.ds(0, CHUNK)], idx_v.at[s], isems.at[s]
        ).wait()
        pltpu.make_async_copy(
            w_hbm.at[idx_v.at[s]], buf_v.at[bg], gsems.at[bg]
        ).start()

    for c in range(PREF2):
        stage(c)
    for g in range(PREF):
        gather(g, g)

    def step(j, carry):
        b = lax.rem(j, NBUF)
        pltpu.make_async_copy(
            w_hbm.at[idx_v.at[0]], buf_v.at[b], gsems.at[b]
        ).wait()
        pltpu.make_async_copy(
            buf_v.at[b], out_flat.at[pl.ds(base + j * CHUNK, CHUNK)], wsems.at[b]
        ).start()
        g = j + PREF

        @pl.when(g < NCHUNK)
        def _():
            bg = lax.rem(g, NBUF)

            @pl.when(g >= NBUF)
            def _():
                pltpu.make_async_copy(
                    buf_v.at[bg],
                    out_flat.at[pl.ds(base + (g - NBUF) * CHUNK, CHUNK)],
                    wsems.at[bg],
                ).wait()

            gather(g, bg)

        c = j + PREF2

        @pl.when(c < NCHUNK)
        def _():
            stage(c)

        return carry

    lax.fori_loop(0, NCHUNK, step, 0)

    for t in range(NBUF):
        j = NCHUNK - NBUF + t
        b = j % NBUF
        pltpu.make_async_copy(
            buf_v.at[b], out_flat.at[pl.ds(base + j * CHUNK, CHUNK)], wsems.at[b]
        ).wait()


@jax.jit
def _run(targ, W):
    tt = targ.T
    mesh = plsc.VectorSubcoreMesh(core_axis_name="c", subcore_axis_name="s")
    k = pl.kernel(
        _body,
        out_type=jax.ShapeDtypeStruct((HIST, BATCH, EMBED_DIM), jnp.float32),
        mesh=mesh,
        compiler_params=pltpu.CompilerParams(use_tc_tiling_on_sc=True),
        scratch_types=[
            pltpu.VMEM((NIDX, CHUNK), jnp.int32),
            pltpu.VMEM((NBUF, CHUNK, EMBED_DIM), jnp.float32),
            pltpu.SemaphoreType.DMA((NIDX,)),
            pltpu.SemaphoreType.DMA((NBUF,)),
            pltpu.SemaphoreType.DMA((NBUF,)),
        ],
    )
    out_t = k(tt, W)
    return jnp.transpose(out_t, (1, 0, 2))


def kernel(targ, W):
    return _run(targ.astype(jnp.int32), W)
